# baseline (speedup 1.0000x reference)
import functools

import jax
import jax.numpy as jnp
from jax import lax
from jax.experimental import pallas as pl
from jax.experimental.pallas import tpu as pltpu
from jax.experimental.pallas import tpu_sc as plsc

DECAY = 0.99
THRESH = 1.0

_PREC = lax.Precision.DEFAULT


def _prep(flat, W_pred, write_idx, strength, BUF, S, blk=256):
    D = flat.shape[1]
    nblk = S // blk
    w_row = write_idx.reshape(1, S)
    w_col = write_idx.reshape(S, 1)
    s_row = strength.reshape(1, BUF)

    def body(x_ref, w_ref, wr_ref, wcb_ref, wcf_ref, s_ref,
             u_ref, dst_ref, msk_ref, b_ref):
        pid = pl.program_id(0)
        xb = x_ref[...]
        pred = jnp.dot(xb, w_ref[...], preferred_element_type=jnp.float32,
                       precision=_PREC)
        surprise = jnp.mean((xb - pred) ** 2, axis=1, keepdims=True)
        gate = jax.nn.sigmoid(surprise - THRESH)
        u_ref[...] = (gate * xb).T
        widx_row = wr_ref[...]
        wblk_col = wcb_ref[...]
        wcol_full = wcf_ref[...]
        tblk_col = pid * blk + lax.broadcasted_iota(jnp.int32, (blk, 1), 0)
        tok_row = lax.broadcasted_iota(jnp.int32, (blk, S), 1)
        dup = (wblk_col == widx_row) & (tok_row > tblk_col)
        winner = jnp.logical_not(jnp.any(dup, axis=1, keepdims=True))
        dst_ref[...] = jnp.where(winner, wblk_col, BUF + tblk_col)
        slots_col = tblk_col
        eq_c = widx_row == slots_col
        msk_ref[...] = jnp.any(eq_c, axis=1, keepdims=True).astype(
            jnp.float32)
        slots_row = pid * blk + lax.broadcasted_iota(jnp.int32, (1, blk), 1)
        eq_r = wcol_full == slots_row
        written_r = jnp.any(eq_r, axis=0, keepdims=True)
        ns = jnp.where(written_r, 1.0, s_ref[...] * DECAY)
        b_ref[...] = jnp.log(ns + 1e-6)

    uT, dst, msk, bias = pl.pallas_call(
        body,
        grid=(nblk,),
        in_specs=[
            pl.BlockSpec((blk, D), lambda i: (i, 0)),
            pl.BlockSpec((D, D), lambda i: (0, 0)),
            pl.BlockSpec((1, S), lambda i: (0, 0)),
            pl.BlockSpec((blk, 1), lambda i: (i, 0)),
            pl.BlockSpec((S, 1), lambda i: (0, 0)),
            pl.BlockSpec((1, blk), lambda i: (0, i)),
        ],
        out_specs=[
            pl.BlockSpec((D, blk), lambda i: (0, i)),
            pl.BlockSpec((blk, 1), lambda i: (i, 0)),
            pl.BlockSpec((blk, 1), lambda i: (i, 0)),
            pl.BlockSpec((1, blk), lambda i: (0, i)),
        ],
        out_shape=[
            jax.ShapeDtypeStruct((D, S), jnp.float32),
            jax.ShapeDtypeStruct((S, 1), jnp.int32),
            jax.ShapeDtypeStruct((BUF, 1), jnp.float32),
            jax.ShapeDtypeStruct((1, BUF), jnp.float32),
        ],
    )(flat, W_pred, w_row, w_col, w_col, s_row)
    return uT, dst.reshape(S), msk, bias


_SC_MESH = dict(core_axis_name="c", subcore_axis_name="s",
                num_cores=2, num_subcores=16)
_SC_PARAMS = dict(needs_layout_passes=False, use_tc_tiling_on_sc=False)


def _sc_scatter_nb(flat, dst_idx, BUF, D, S):
    NC, NS = 2, 16
    NW = NC * NS
    TPW = S // NW
    TSUB = 16

    @functools.partial(
        pl.kernel,
        out_type=jax.ShapeDtypeStruct((2 * BUF, D), jnp.float32),
        mesh=plsc.VectorSubcoreMesh(**_SC_MESH),
        compiler_params=pltpu.CompilerParams(needs_layout_passes=False),
        scratch_types=(
            pltpu.VMEM((TSUB,), jnp.int32),
            pltpu.VMEM((TSUB, D), jnp.float32),
            pltpu.SemaphoreType.DMA,
        ),
    )
    def body(flat_h, didx_h, nb_h, didx_v, rows_v, sem):
        c = lax.axis_index("c")
        s = lax.axis_index("s")
        wid = s * NC + c
        for g in range(TPW // TSUB):
            tb = wid * TPW + g * TSUB
            pltpu.sync_copy(flat_h.at[pl.ds(tb, TSUB)], rows_v)
            pltpu.sync_copy(didx_h.at[pl.ds(tb, TSUB)], didx_v)
            pltpu.async_copy(rows_v, nb_h.at[didx_v], sem).wait()

    return body(flat, dst_idx)


def _sc_delta(uT, store_idx, zeros, STORE, D, S):
    NC, NS = 2, 16
    NW = NC * NS
    NCH = D // 16
    CPW = NCH // NW

    @functools.partial(
        pl.kernel,
        out_type=jax.ShapeDtypeStruct((STORE, D), jnp.float32),
        mesh=plsc.VectorSubcoreMesh(**_SC_MESH),
        compiler_params=pltpu.CompilerParams(**_SC_PARAMS),
        scratch_types=(
            pltpu.VMEM((S,), jnp.int32),
            pltpu.VMEM((16, S), jnp.float32),
            pltpu.VMEM((STORE, 16), jnp.float32),
        ),
    )
    def body(u_h, sidx_h, zeros_h, delta_h, sidx_v, u_v, acc_v):
        c = lax.axis_index("c")
        s = lax.axis_index("s")
        wid = s * NC + c
        pltpu.sync_copy(sidx_h, sidx_v)
        for p in range(CPW):
            ch = wid * CPW + p
            pltpu.sync_copy(zeros_h, acc_v)
            pltpu.sync_copy(u_h.at[pl.ds(ch * 16, 16)], u_v)

            def tok_body(g, carry):
                tb = g * 16
                st16 = sidx_v[pl.ds(tb, 16)]
                for col in range(16):
                    vals = u_v[col, pl.ds(tb, 16)]
                    plsc.addupdate_scatter(
                        acc_v, [st16, jnp.full((16,), col, jnp.int32)], vals)
                return carry

            lax.fori_loop(0, S // 16, tok_body, 0)
            pltpu.sync_copy(acc_v, delta_h.at[:, pl.ds(ch * 16, 16)])

    return body(uT, store_idx, zeros)


def _buffer_attn(flat, nb2, buffer_mem, msk, bias2, blk=256):
    S, D = flat.shape
    BUF = buffer_mem.shape[0]
    scale = 1.0 / (D ** 0.5)

    def body(x_ref, nb_ref, bm_ref, m_ref, b_ref, o_ref):
        xb = x_ref[...]
        nbv = jnp.where(m_ref[...] > 0.0, nb_ref[...], bm_ref[...])
        logits = lax.dot_general(xb, nbv, (((1,), (1,)), ((), ())),
                                 preferred_element_type=jnp.float32,
                                 precision=_PREC) * scale + b_ref[...]
        m = jnp.max(logits, axis=1, keepdims=True)
        p = jnp.exp(logits - m)
        attn = p / jnp.sum(p, axis=1, keepdims=True)
        o_ref[...] = jnp.dot(attn, nbv, preferred_element_type=jnp.float32,
                             precision=_PREC)

    return pl.pallas_call(
        body,
        grid=(S // blk,),
        in_specs=[
            pl.BlockSpec((blk, D), lambda i: (i, 0)),
            pl.BlockSpec((BUF, D), lambda i: (0, 0)),
            pl.BlockSpec((BUF, D), lambda i: (0, 0)),
            pl.BlockSpec((BUF, 1), lambda i: (0, 0)),
            pl.BlockSpec((1, BUF), lambda i: (0, 0)),
        ],
        out_specs=pl.BlockSpec((blk, D), lambda i: (i, 0)),
        out_shape=jax.ShapeDtypeStruct((S, D), jnp.float32),
    )(flat, nb2, buffer_mem, msk, bias2)


def _store_attn_mlp(flat, keys, values, delta, br, W_g1, b_g1, w20, w21,
                    b_g2, W_out, b_out, ln_g, ln_b, kblk=256):
    S, D = flat.shape
    STORE = keys.shape[0]
    scale = 1.0 / (D ** 0.5)
    nk = STORE // kblk

    def body(x_ref, k_ref, v_ref, d_ref, br_ref, w1_ref, b1_ref, w20_ref,
             w21_ref, b2_ref, wo_ref, bo_ref, lg_ref, lb_ref, o_ref,
             m_ref, l_ref):
        k = pl.program_id(0)

        @pl.when(k == 0)
        def _():
            m_ref[...] = jnp.full((S, 128), -1e30, jnp.float32)
            l_ref[...] = jnp.zeros((S, 128), jnp.float32)
            o_ref[...] = jnp.zeros((S, D), jnp.float32)

        xb = x_ref[...]
        dlt = d_ref[...]
        kk = k_ref[...] + dlt
        s = lax.dot_general(xb, kk, (((1,), (1,)), ((), ())),
                            preferred_element_type=jnp.float32,
                            precision=_PREC) * scale
        m_old = m_ref[...][:, :1]
        m_new = jnp.maximum(m_old, jnp.max(s, axis=1, keepdims=True))
        alpha = jnp.exp(m_old - m_new)
        p = jnp.exp(s - m_new)
        l_new = l_ref[...][:, :1] * alpha + jnp.sum(p, axis=1, keepdims=True)
        o_ref[...] = o_ref[...] * alpha + jnp.dot(
            p, v_ref[...] + dlt, preferred_element_type=jnp.float32,
            precision=_PREC)
        m_ref[...] = jnp.broadcast_to(m_new, (S, 128))
        l_ref[...] = jnp.broadcast_to(l_new, (S, 128))

        @pl.when(k == nk - 1)
        def _():
            cblk = 256

            def mlp_chunk(i, carry):
                sl = pl.ds(i * cblk, cblk)
                xc = x_ref[sl, :]
                brv = br_ref[sl, :]
                srv = o_ref[sl, :] / l_ref[sl, :][:, :1]
                h = (jnp.dot(xc, w1_ref[0:D, :],
                             preferred_element_type=jnp.float32,
                             precision=_PREC)
                     + jnp.dot(brv, w1_ref[D:2 * D, :],
                               preferred_element_type=jnp.float32,
                               precision=_PREC)
                     + jnp.dot(srv, w1_ref[2 * D:3 * D, :],
                               preferred_element_type=jnp.float32,
                               precision=_PREC)
                     + b1_ref[...])
                h = jax.nn.gelu(h)
                g0 = jax.nn.sigmoid(
                    jnp.sum(h * w20_ref[...], axis=1, keepdims=True)
                    + b2_ref[0])
                g1 = jax.nn.sigmoid(
                    jnp.sum(h * w21_ref[...], axis=1, keepdims=True)
                    + b2_ref[1])
                mc = g0 * brv + g1 * srv
                mu = jnp.mean(mc, axis=1, keepdims=True)
                var = jnp.mean((mc - mu) ** 2, axis=1, keepdims=True)
                normed = ((mc - mu) * lax.rsqrt(var + 1e-5) * lg_ref[...]
                          + lb_ref[...])
                o_ref[sl, :] = xc + jnp.dot(
                    normed, wo_ref[...], preferred_element_type=jnp.float32,
                    precision=_PREC) + bo_ref[...]
                return carry

            lax.fori_loop(0, S // cblk, mlp_chunk, 0)

    return pl.pallas_call(
        body,
        grid=(nk,),
        in_specs=[
            pl.BlockSpec((S, D), lambda j: (0, 0)),
            pl.BlockSpec((kblk, D), lambda j: (j, 0)),
            pl.BlockSpec((kblk, D), lambda j: (j, 0)),
            pl.BlockSpec((kblk, D), lambda j: (j, 0)),
            pl.BlockSpec((S, D), lambda j: (0, 0)),
            pl.BlockSpec((3 * D, D), lambda j: (0, 0)),
            pl.BlockSpec((1, D), lambda j: (0, 0)),
            pl.BlockSpec((1, D), lambda j: (0, 0)),
            pl.BlockSpec((1, D), lambda j: (0, 0)),
            pl.BlockSpec(memory_space=pltpu.SMEM),
            pl.BlockSpec((D, D), lambda j: (0, 0)),
            pl.BlockSpec((1, D), lambda j: (0, 0)),
            pl.BlockSpec((1, D), lambda j: (0, 0)),
            pl.BlockSpec((1, D), lambda j: (0, 0)),
        ],
        out_specs=pl.BlockSpec((S, D), lambda j: (0, 0)),
        out_shape=jax.ShapeDtypeStruct((S, D), jnp.float32),
        scratch_shapes=[
            pltpu.VMEM((S, 128), jnp.float32),
            pltpu.VMEM((S, 128), jnp.float32),
        ],
    )(flat, keys, values, delta, br, W_g1, b_g1, w20, w21, b_g2, W_out,
      b_out, ln_g, ln_b)


def kernel(x, W_pred, buffer_mem, buffer_strength, store_keys, store_values,
           W_g1, b_g1, W_g2, b_g2, W_out, b_out, ln_g, ln_b,
           write_idx, store_idx):
    B, S, D = x.shape
    BUF = buffer_mem.shape[0]
    STORE = store_keys.shape[0]
    NCH = D // 16

    flat = x.reshape(S, D)
    write_idx = write_idx.astype(jnp.int32)
    store_idx = store_idx.astype(jnp.int32)

    uT, dst_idx, msk, bias = _prep(flat, W_pred, write_idx,
                                   buffer_strength, BUF, S)
    zeros = jnp.zeros((STORE, 16), jnp.float32)
    nb2 = _sc_scatter_nb(flat, dst_idx, BUF, D, S)
    delta = _sc_delta(uT, store_idx, zeros, STORE, D, S)
    br = _buffer_attn(flat, nb2, buffer_mem, msk, bias)
    out = _store_attn_mlp(flat, store_keys, store_values, delta, br,
                          W_g1, b_g1.reshape(1, D),
                          W_g2[:, 0].reshape(1, D), W_g2[:, 1].reshape(1, D),
                          b_g2, W_out, b_out.reshape(1, D),
                          ln_g.reshape(1, D), ln_b.reshape(1, D))
    return out.reshape(B, S, D)


# speedup vs baseline: 1.1107x; 1.1107x over previous
import functools

import jax
import jax.numpy as jnp
from jax import lax
from jax.experimental import pallas as pl
from jax.experimental.pallas import tpu as pltpu
from jax.experimental.pallas import tpu_sc as plsc

DECAY = 0.99
THRESH = 1.0

_PREC = lax.Precision.DEFAULT


def _prep(flat, W_pred, write_idx, strength, BUF, S, blk=256):
    D = flat.shape[1]
    nblk = S // blk
    w_row = write_idx.reshape(1, S)
    w_col = write_idx.reshape(S, 1)
    s_row = strength.reshape(1, BUF)

    def body(x_ref, w_ref, wr_ref, wcb_ref, wcf_ref, s_ref,
             u_ref, dst_ref, msk_ref, b_ref):
        pid = pl.program_id(0)
        xb = x_ref[...]
        pred = jnp.dot(xb, w_ref[...], preferred_element_type=jnp.float32,
                       precision=_PREC)
        surprise = jnp.mean((xb - pred) ** 2, axis=1, keepdims=True)
        gate = jax.nn.sigmoid(surprise - THRESH)
        u_ref[...] = (gate * xb).T
        widx_row = wr_ref[...]
        wblk_col = wcb_ref[...]
        wcol_full = wcf_ref[...]
        tblk_col = pid * blk + lax.broadcasted_iota(jnp.int32, (blk, 1), 0)
        tok_row = lax.broadcasted_iota(jnp.int32, (blk, S), 1)
        dup = (wblk_col == widx_row) & (tok_row > tblk_col)
        winner = jnp.logical_not(jnp.any(dup, axis=1, keepdims=True))
        dst_ref[...] = jnp.where(winner, wblk_col, BUF + tblk_col)
        slots_col = tblk_col
        eq_c = widx_row == slots_col
        msk_ref[...] = jnp.any(eq_c, axis=1, keepdims=True).astype(
            jnp.float32)
        slots_row = pid * blk + lax.broadcasted_iota(jnp.int32, (1, blk), 1)
        eq_r = wcol_full == slots_row
        written_r = jnp.any(eq_r, axis=0, keepdims=True)
        ns = jnp.where(written_r, 1.0, s_ref[...] * DECAY)
        b_ref[...] = jnp.log(ns + 1e-6)

    uT, dst, msk, bias = pl.pallas_call(
        body,
        grid=(nblk,),
        in_specs=[
            pl.BlockSpec((blk, D), lambda i: (i, 0)),
            pl.BlockSpec((D, D), lambda i: (0, 0)),
            pl.BlockSpec((1, S), lambda i: (0, 0)),
            pl.BlockSpec((blk, 1), lambda i: (i, 0)),
            pl.BlockSpec((S, 1), lambda i: (0, 0)),
            pl.BlockSpec((1, blk), lambda i: (0, i)),
        ],
        out_specs=[
            pl.BlockSpec((D, blk), lambda i: (0, i)),
            pl.BlockSpec((blk, 1), lambda i: (i, 0)),
            pl.BlockSpec((blk, 1), lambda i: (i, 0)),
            pl.BlockSpec((1, blk), lambda i: (0, i)),
        ],
        out_shape=[
            jax.ShapeDtypeStruct((D, S), jnp.float32),
            jax.ShapeDtypeStruct((S, 1), jnp.int32),
            jax.ShapeDtypeStruct((BUF, 1), jnp.float32),
            jax.ShapeDtypeStruct((1, BUF), jnp.float32),
        ],
    )(flat, W_pred, w_row, w_col, w_col, s_row)
    return uT, dst.reshape(S), msk, bias


_SC_MESH = dict(core_axis_name="c", subcore_axis_name="s",
                num_cores=2, num_subcores=16)
_SC_PARAMS = dict(needs_layout_passes=False, use_tc_tiling_on_sc=False)


def _sc_scatter_nb(flat, dst_idx, BUF, D, S):
    NC, NS = 2, 16
    NW = NC * NS
    TPW = S // NW
    TSUB = 16

    @functools.partial(
        pl.kernel,
        out_type=jax.ShapeDtypeStruct((2 * BUF, D), jnp.float32),
        mesh=plsc.VectorSubcoreMesh(**_SC_MESH),
        compiler_params=pltpu.CompilerParams(needs_layout_passes=False),
        scratch_types=(
            pltpu.VMEM((TSUB,), jnp.int32),
            pltpu.VMEM((TSUB, D), jnp.float32),
            pltpu.SemaphoreType.DMA,
        ),
    )
    def body(flat_h, didx_h, nb_h, didx_v, rows_v, sem):
        c = lax.axis_index("c")
        s = lax.axis_index("s")
        wid = s * NC + c
        for g in range(TPW // TSUB):
            tb = wid * TPW + g * TSUB
            pltpu.sync_copy(flat_h.at[pl.ds(tb, TSUB)], rows_v)
            pltpu.sync_copy(didx_h.at[pl.ds(tb, TSUB)], didx_v)
            pltpu.async_copy(rows_v, nb_h.at[didx_v], sem).wait()

    return body(flat, dst_idx)


def _sc_delta(uT, store_idx, zeros, STORE, D, S):
    NC, NS = 2, 16
    NW = NC * NS
    NCH = D // 16
    CPW = NCH // NW

    @functools.partial(
        pl.kernel,
        out_type=jax.ShapeDtypeStruct((STORE, D), jnp.float32),
        mesh=plsc.VectorSubcoreMesh(**_SC_MESH),
        compiler_params=pltpu.CompilerParams(**_SC_PARAMS),
        scratch_types=(
            pltpu.VMEM((S,), jnp.int32),
            pltpu.VMEM((16, S), jnp.float32),
            pltpu.VMEM((STORE, 16), jnp.float32),
        ),
    )
    def body(u_h, sidx_h, zeros_h, delta_h, sidx_v, u_v, acc_v):
        c = lax.axis_index("c")
        s = lax.axis_index("s")
        wid = s * NC + c
        pltpu.sync_copy(sidx_h, sidx_v)
        for p in range(CPW):
            ch = wid * CPW + p
            pltpu.sync_copy(zeros_h, acc_v)
            pltpu.sync_copy(u_h.at[pl.ds(ch * 16, 16)], u_v)

            def tok_body(g, carry):
                tb = g * 16
                st16 = sidx_v[pl.ds(tb, 16)]
                for col in range(16):
                    vals = u_v[col, pl.ds(tb, 16)]
                    plsc.addupdate_scatter(
                        acc_v, [st16, jnp.full((16,), col, jnp.int32)], vals)
                return carry

            lax.fori_loop(0, S // 16, tok_body, 0)
            pltpu.sync_copy(acc_v, delta_h.at[:, pl.ds(ch * 16, 16)])

    return body(uT, store_idx, zeros)


def _buffer_attn(flat, nb2, buffer_mem, msk, bias2, blk=256):
    S, D = flat.shape
    BUF = buffer_mem.shape[0]
    scale = 1.0 / (D ** 0.5)

    def body(x_ref, nb_ref, bm_ref, m_ref, b_ref, o_ref):
        xb = x_ref[...]
        nbv = jnp.where(m_ref[...] > 0.0, nb_ref[...], bm_ref[...])
        logits = lax.dot_general(xb, nbv, (((1,), (1,)), ((), ())),
                                 preferred_element_type=jnp.float32,
                                 precision=_PREC) * scale + b_ref[...]
        m = jnp.max(logits, axis=1, keepdims=True)
        p = jnp.exp(logits - m)
        attn = p / jnp.sum(p, axis=1, keepdims=True)
        o_ref[...] = jnp.dot(attn, nbv, preferred_element_type=jnp.float32,
                             precision=_PREC)

    return pl.pallas_call(
        body,
        grid=(S // blk,),
        in_specs=[
            pl.BlockSpec((blk, D), lambda i: (i, 0)),
            pl.BlockSpec((BUF, D), lambda i: (0, 0)),
            pl.BlockSpec((BUF, D), lambda i: (0, 0)),
            pl.BlockSpec((BUF, 1), lambda i: (0, 0)),
            pl.BlockSpec((1, BUF), lambda i: (0, 0)),
        ],
        out_specs=pl.BlockSpec((blk, D), lambda i: (i, 0)),
        out_shape=jax.ShapeDtypeStruct((S, D), jnp.float32),
    )(flat, nb2, buffer_mem, msk, bias2)


def _store_attn(flat, keys, values, delta, kblk=512):
    S, D = flat.shape
    STORE = keys.shape[0]
    scale = 1.0 / (D ** 0.5)
    nk = STORE // kblk

    def body(x_ref, k_ref, v_ref, d_ref, o_ref, acc_ref, m_ref, l_ref):
        k = pl.program_id(0)

        @pl.when(k == 0)
        def _():
            m_ref[...] = jnp.full((S, 128), -1e30, jnp.float32)
            l_ref[...] = jnp.zeros((S, 128), jnp.float32)
            acc_ref[...] = jnp.zeros((S, D), jnp.float32)

        xb = x_ref[...]
        dlt = d_ref[...]
        kk = k_ref[...] + dlt
        s = lax.dot_general(xb, kk, (((1,), (1,)), ((), ())),
                            preferred_element_type=jnp.float32,
                            precision=_PREC) * scale
        m_old = m_ref[...][:, :1]
        m_new = jnp.maximum(m_old, jnp.max(s, axis=1, keepdims=True))
        alpha = jnp.exp(m_old - m_new)
        p = jnp.exp(s - m_new)
        l_new = l_ref[...][:, :1] * alpha + jnp.sum(p, axis=1, keepdims=True)
        acc_ref[...] = acc_ref[...] * alpha + jnp.dot(
            p, v_ref[...] + dlt, preferred_element_type=jnp.float32,
            precision=_PREC)
        m_ref[...] = jnp.broadcast_to(m_new, (S, 128))
        l_ref[...] = jnp.broadcast_to(l_new, (S, 128))

        @pl.when(k == nk - 1)
        def _():
            o_ref[...] = acc_ref[...] / l_ref[...][:, :1]

    return pl.pallas_call(
        body,
        grid=(nk,),
        in_specs=[
            pl.BlockSpec((S, D), lambda j: (0, 0)),
            pl.BlockSpec((kblk, D), lambda j: (j, 0)),
            pl.BlockSpec((kblk, D), lambda j: (j, 0)),
            pl.BlockSpec((kblk, D), lambda j: (j, 0)),
        ],
        out_specs=pl.BlockSpec((S, D), lambda j: (0, 0)),
        out_shape=jax.ShapeDtypeStruct((S, D), jnp.float32),
        scratch_shapes=[
            pltpu.VMEM((S, D), jnp.float32),
            pltpu.VMEM((S, 128), jnp.float32),
            pltpu.VMEM((S, 128), jnp.float32),
        ],
    )(flat, keys, values, delta)


def _mlp_out(flat, br, sr, W_g1, b_g1, w20, w21, b_g2, W_out, b_out,
             ln_g, ln_b, blk=256):
    S, D = flat.shape

    def body(x_ref, br_ref, sr_ref, w1_ref, b1_ref, w20_ref, w21_ref,
             b2_ref, wo_ref, bo_ref, lg_ref, lb_ref, o_ref):
        xb = x_ref[...]
        brv = br_ref[...]
        srv = sr_ref[...]
        h = (jnp.dot(xb, w1_ref[0:D, :], preferred_element_type=jnp.float32,
                     precision=_PREC)
             + jnp.dot(brv, w1_ref[D:2 * D, :],
                       preferred_element_type=jnp.float32, precision=_PREC)
             + jnp.dot(srv, w1_ref[2 * D:3 * D, :],
                       preferred_element_type=jnp.float32, precision=_PREC)
             + b1_ref[...])
        h = jax.nn.gelu(h)
        g0 = jax.nn.sigmoid(
            jnp.sum(h * w20_ref[...], axis=1, keepdims=True) + b2_ref[0])
        g1 = jax.nn.sigmoid(
            jnp.sum(h * w21_ref[...], axis=1, keepdims=True) + b2_ref[1])
        mc = g0 * brv + g1 * srv
        mu = jnp.mean(mc, axis=1, keepdims=True)
        var = jnp.mean((mc - mu) ** 2, axis=1, keepdims=True)
        normed = (mc - mu) * lax.rsqrt(var + 1e-5) * lg_ref[...] + lb_ref[...]
        o_ref[...] = xb + jnp.dot(normed, wo_ref[...],
                                  preferred_element_type=jnp.float32,
                                  precision=_PREC) + bo_ref[...]

    return pl.pallas_call(
        body,
        grid=(S // blk,),
        in_specs=[
            pl.BlockSpec((blk, D), lambda i: (i, 0)),
            pl.BlockSpec((blk, D), lambda i: (i, 0)),
            pl.BlockSpec((blk, D), lambda i: (i, 0)),
            pl.BlockSpec((3 * D, D), lambda i: (0, 0)),
            pl.BlockSpec((1, D), lambda i: (0, 0)),
            pl.BlockSpec((1, D), lambda i: (0, 0)),
            pl.BlockSpec((1, D), lambda i: (0, 0)),
            pl.BlockSpec(memory_space=pltpu.SMEM),
            pl.BlockSpec((D, D), lambda i: (0, 0)),
            pl.BlockSpec((1, D), lambda i: (0, 0)),
            pl.BlockSpec((1, D), lambda i: (0, 0)),
            pl.BlockSpec((1, D), lambda i: (0, 0)),
        ],
        out_specs=pl.BlockSpec((blk, D), lambda i: (i, 0)),
        out_shape=jax.ShapeDtypeStruct((S, D), jnp.float32),
    )(flat, br, sr, W_g1, b_g1, w20, w21, b_g2, W_out, b_out, ln_g, ln_b)


def kernel(x, W_pred, buffer_mem, buffer_strength, store_keys, store_values,
           W_g1, b_g1, W_g2, b_g2, W_out, b_out, ln_g, ln_b,
           write_idx, store_idx):
    B, S, D = x.shape
    BUF = buffer_mem.shape[0]
    STORE = store_keys.shape[0]
    NCH = D // 16

    flat = x.reshape(S, D)
    write_idx = write_idx.astype(jnp.int32)
    store_idx = store_idx.astype(jnp.int32)

    uT, dst_idx, msk, bias = _prep(flat, W_pred, write_idx,
                                   buffer_strength, BUF, S)
    zeros = jnp.zeros((STORE, 16), jnp.float32)
    nb2 = _sc_scatter_nb(flat, dst_idx, BUF, D, S)
    delta = _sc_delta(uT, store_idx, zeros, STORE, D, S)
    br = _buffer_attn(flat, nb2, buffer_mem, msk, bias)
    sr = _store_attn(flat, store_keys, store_values, delta)
    out = _mlp_out(flat, br, sr, W_g1, b_g1.reshape(1, D),
                   W_g2[:, 0].reshape(1, D), W_g2[:, 1].reshape(1, D),
                   b_g2, W_out, b_out.reshape(1, D), ln_g.reshape(1, D),
                   ln_b.reshape(1, D))
    return out.reshape(B, S, D)


# speedup vs baseline: 1.1549x; 1.0398x over previous
import functools

import jax
import jax.numpy as jnp
from jax import lax
from jax.experimental import pallas as pl
from jax.experimental.pallas import tpu as pltpu
from jax.experimental.pallas import tpu_sc as plsc

DECAY = 0.99
THRESH = 1.0

_PREC = lax.Precision.DEFAULT


def _prep(flat, W_pred, write_idx, strength, BUF, S, blk=256):
    D = flat.shape[1]
    nblk = S // blk
    w_row = write_idx.reshape(1, S)
    w_col = write_idx.reshape(S, 1)
    s_row = strength.reshape(1, BUF)

    def body(x_ref, w_ref, wr_ref, wcb_ref, wcf_ref, s_ref,
             u_ref, dst_ref, msk_ref, b_ref):
        pid = pl.program_id(0)
        xb = x_ref[...]
        pred = jnp.dot(xb, w_ref[...], preferred_element_type=jnp.float32,
                       precision=_PREC)
        surprise = jnp.mean((xb - pred) ** 2, axis=1, keepdims=True)
        gate = jax.nn.sigmoid(surprise - THRESH)
        u_ref[...] = (gate * xb).T
        widx_row = wr_ref[...]
        wblk_col = wcb_ref[...]
        wcol_full = wcf_ref[...]
        tblk_col = pid * blk + lax.broadcasted_iota(jnp.int32, (blk, 1), 0)
        tok_row = lax.broadcasted_iota(jnp.int32, (blk, S), 1)
        dup = (wblk_col == widx_row) & (tok_row > tblk_col)
        winner = jnp.logical_not(jnp.any(dup, axis=1, keepdims=True))
        dst_ref[...] = jnp.where(winner, wblk_col, BUF + tblk_col)
        slots_col = tblk_col
        eq_c = widx_row == slots_col
        msk_ref[...] = jnp.any(eq_c, axis=1, keepdims=True).astype(
            jnp.float32)
        slots_row = pid * blk + lax.broadcasted_iota(jnp.int32, (1, blk), 1)
        eq_r = wcol_full == slots_row
        written_r = jnp.any(eq_r, axis=0, keepdims=True)
        ns = jnp.where(written_r, 1.0, s_ref[...] * DECAY)
        b_ref[...] = jnp.log(ns + 1e-6)

    uT, dst, msk, bias = pl.pallas_call(
        body,
        grid=(nblk,),
        in_specs=[
            pl.BlockSpec((blk, D), lambda i: (i, 0)),
            pl.BlockSpec((D, D), lambda i: (0, 0)),
            pl.BlockSpec((1, S), lambda i: (0, 0)),
            pl.BlockSpec((blk, 1), lambda i: (i, 0)),
            pl.BlockSpec((S, 1), lambda i: (0, 0)),
            pl.BlockSpec((1, blk), lambda i: (0, i)),
        ],
        out_specs=[
            pl.BlockSpec((D, blk), lambda i: (0, i)),
            pl.BlockSpec((blk, 1), lambda i: (i, 0)),
            pl.BlockSpec((blk, 1), lambda i: (i, 0)),
            pl.BlockSpec((1, blk), lambda i: (0, i)),
        ],
        out_shape=[
            jax.ShapeDtypeStruct((D, S), jnp.float32),
            jax.ShapeDtypeStruct((S, 1), jnp.int32),
            jax.ShapeDtypeStruct((BUF, 1), jnp.float32),
            jax.ShapeDtypeStruct((1, BUF), jnp.float32),
        ],
    )(flat, W_pred, w_row, w_col, w_col, s_row)
    return uT, dst.reshape(S), msk, bias


_SC_MESH = dict(core_axis_name="c", subcore_axis_name="s",
                num_cores=2, num_subcores=16)
_SC_PARAMS = dict(needs_layout_passes=False, use_tc_tiling_on_sc=False)


def _sc_scatter_nb(flat, dst_idx, BUF, D, S):
    NC, NS = 2, 16
    NW = NC * NS
    TPW = S // NW
    TSUB = 16

    @functools.partial(
        pl.kernel,
        out_type=jax.ShapeDtypeStruct((2 * BUF, D), jnp.float32),
        mesh=plsc.VectorSubcoreMesh(**_SC_MESH),
        compiler_params=pltpu.CompilerParams(needs_layout_passes=False),
        scratch_types=(
            pltpu.VMEM((TSUB,), jnp.int32),
            pltpu.VMEM((TSUB, D), jnp.float32),
            pltpu.SemaphoreType.DMA,
        ),
    )
    def body(flat_h, didx_h, nb_h, didx_v, rows_v, sem):
        c = lax.axis_index("c")
        s = lax.axis_index("s")
        wid = s * NC + c
        for g in range(TPW // TSUB):
            tb = wid * TPW + g * TSUB
            pltpu.sync_copy(flat_h.at[pl.ds(tb, TSUB)], rows_v)
            pltpu.sync_copy(didx_h.at[pl.ds(tb, TSUB)], didx_v)
            pltpu.async_copy(rows_v, nb_h.at[didx_v], sem).wait()

    return body(flat, dst_idx)


def _sc_delta(uT, store_idx, zeros, STORE, D, S):
    NC, NS = 2, 16
    NW = NC * NS
    NCH = D // 16
    CPW = NCH // NW

    @functools.partial(
        pl.kernel,
        out_type=jax.ShapeDtypeStruct((STORE, D), jnp.float32),
        mesh=plsc.VectorSubcoreMesh(**_SC_MESH),
        compiler_params=pltpu.CompilerParams(**_SC_PARAMS),
        scratch_types=(
            pltpu.VMEM((S,), jnp.int32),
            pltpu.VMEM((16, S), jnp.float32),
            pltpu.VMEM((STORE, 16), jnp.float32),
        ),
    )
    def body(u_h, sidx_h, zeros_h, delta_h, sidx_v, u_v, acc_v):
        c = lax.axis_index("c")
        s = lax.axis_index("s")
        wid = s * NC + c
        pltpu.sync_copy(sidx_h, sidx_v)
        for p in range(CPW):
            ch = wid * CPW + p
            pltpu.sync_copy(zeros_h, acc_v)
            pltpu.sync_copy(u_h.at[pl.ds(ch * 16, 16)], u_v)

            @plsc.parallel_loop(0, S // 16, unroll=4)
            def tok_body(g):
                tb = g * 16
                st16 = sidx_v[pl.ds(tb, 16)]
                for col in range(16):
                    vals = u_v[col, pl.ds(tb, 16)]
                    plsc.addupdate_scatter(
                        acc_v, [st16, jnp.full((16,), col, jnp.int32)], vals)
            pltpu.sync_copy(acc_v, delta_h.at[:, pl.ds(ch * 16, 16)])

    return body(uT, store_idx, zeros)


def _buffer_attn(flat, nb2, buffer_mem, msk, bias2, blk=256):
    S, D = flat.shape
    BUF = buffer_mem.shape[0]
    scale = 1.0 / (D ** 0.5)

    def body(x_ref, nb_ref, bm_ref, m_ref, b_ref, o_ref):
        xb = x_ref[...]
        nbv = jnp.where(m_ref[...] > 0.0, nb_ref[...], bm_ref[...])
        logits = lax.dot_general(xb, nbv, (((1,), (1,)), ((), ())),
                                 preferred_element_type=jnp.float32,
                                 precision=_PREC) * scale + b_ref[...]
        m = jnp.max(logits, axis=1, keepdims=True)
        p = jnp.exp(logits - m)
        attn = p / jnp.sum(p, axis=1, keepdims=True)
        o_ref[...] = jnp.dot(attn, nbv, preferred_element_type=jnp.float32,
                             precision=_PREC)

    return pl.pallas_call(
        body,
        grid=(S // blk,),
        in_specs=[
            pl.BlockSpec((blk, D), lambda i: (i, 0)),
            pl.BlockSpec((BUF, D), lambda i: (0, 0)),
            pl.BlockSpec((BUF, D), lambda i: (0, 0)),
            pl.BlockSpec((BUF, 1), lambda i: (0, 0)),
            pl.BlockSpec((1, BUF), lambda i: (0, 0)),
        ],
        out_specs=pl.BlockSpec((blk, D), lambda i: (i, 0)),
        out_shape=jax.ShapeDtypeStruct((S, D), jnp.float32),
    )(flat, nb2, buffer_mem, msk, bias2)


def _store_attn(flat, keys, values, delta, kblk=512):
    S, D = flat.shape
    STORE = keys.shape[0]
    scale = 1.0 / (D ** 0.5)
    nk = STORE // kblk

    def body(x_ref, k_ref, v_ref, d_ref, o_ref, acc_ref, m_ref, l_ref):
        k = pl.program_id(0)

        @pl.when(k == 0)
        def _():
            m_ref[...] = jnp.full((S, 128), -1e30, jnp.float32)
            l_ref[...] = jnp.zeros((S, 128), jnp.float32)
            acc_ref[...] = jnp.zeros((S, D), jnp.float32)

        xb = x_ref[...]
        dlt = d_ref[...]
        kk = k_ref[...] + dlt
        s = lax.dot_general(xb, kk, (((1,), (1,)), ((), ())),
                            preferred_element_type=jnp.float32,
                            precision=_PREC) * scale
        m_old = m_ref[...][:, :1]
        m_new = jnp.maximum(m_old, jnp.max(s, axis=1, keepdims=True))
        alpha = jnp.exp(m_old - m_new)
        p = jnp.exp(s - m_new)
        l_new = l_ref[...][:, :1] * alpha + jnp.sum(p, axis=1, keepdims=True)
        acc_ref[...] = acc_ref[...] * alpha + jnp.dot(
            p, v_ref[...] + dlt, preferred_element_type=jnp.float32,
            precision=_PREC)
        m_ref[...] = jnp.broadcast_to(m_new, (S, 128))
        l_ref[...] = jnp.broadcast_to(l_new, (S, 128))

        @pl.when(k == nk - 1)
        def _():
            o_ref[...] = acc_ref[...] / l_ref[...][:, :1]

    return pl.pallas_call(
        body,
        grid=(nk,),
        in_specs=[
            pl.BlockSpec((S, D), lambda j: (0, 0)),
            pl.BlockSpec((kblk, D), lambda j: (j, 0)),
            pl.BlockSpec((kblk, D), lambda j: (j, 0)),
            pl.BlockSpec((kblk, D), lambda j: (j, 0)),
        ],
        out_specs=pl.BlockSpec((S, D), lambda j: (0, 0)),
        out_shape=jax.ShapeDtypeStruct((S, D), jnp.float32),
        scratch_shapes=[
            pltpu.VMEM((S, D), jnp.float32),
            pltpu.VMEM((S, 128), jnp.float32),
            pltpu.VMEM((S, 128), jnp.float32),
        ],
    )(flat, keys, values, delta)


def _mlp_out(flat, br, sr, W_g1, b_g1, w20, w21, b_g2, W_out, b_out,
             ln_g, ln_b, blk=256):
    S, D = flat.shape

    def body(x_ref, br_ref, sr_ref, w1_ref, b1_ref, w20_ref, w21_ref,
             b2_ref, wo_ref, bo_ref, lg_ref, lb_ref, o_ref):
        xb = x_ref[...]
        brv = br_ref[...]
        srv = sr_ref[...]
        h = (jnp.dot(xb, w1_ref[0:D, :], preferred_element_type=jnp.float32,
                     precision=_PREC)
             + jnp.dot(brv, w1_ref[D:2 * D, :],
                       preferred_element_type=jnp.float32, precision=_PREC)
             + jnp.dot(srv, w1_ref[2 * D:3 * D, :],
                       preferred_element_type=jnp.float32, precision=_PREC)
             + b1_ref[...])
        h = jax.nn.gelu(h)
        g0 = jax.nn.sigmoid(
            jnp.sum(h * w20_ref[...], axis=1, keepdims=True) + b2_ref[0])
        g1 = jax.nn.sigmoid(
            jnp.sum(h * w21_ref[...], axis=1, keepdims=True) + b2_ref[1])
        mc = g0 * brv + g1 * srv
        mu = jnp.mean(mc, axis=1, keepdims=True)
        var = jnp.mean((mc - mu) ** 2, axis=1, keepdims=True)
        normed = (mc - mu) * lax.rsqrt(var + 1e-5) * lg_ref[...] + lb_ref[...]
        o_ref[...] = xb + jnp.dot(normed, wo_ref[...],
                                  preferred_element_type=jnp.float32,
                                  precision=_PREC) + bo_ref[...]

    return pl.pallas_call(
        body,
        grid=(S // blk,),
        in_specs=[
            pl.BlockSpec((blk, D), lambda i: (i, 0)),
            pl.BlockSpec((blk, D), lambda i: (i, 0)),
            pl.BlockSpec((blk, D), lambda i: (i, 0)),
            pl.BlockSpec((3 * D, D), lambda i: (0, 0)),
            pl.BlockSpec((1, D), lambda i: (0, 0)),
            pl.BlockSpec((1, D), lambda i: (0, 0)),
            pl.BlockSpec((1, D), lambda i: (0, 0)),
            pl.BlockSpec(memory_space=pltpu.SMEM),
            pl.BlockSpec((D, D), lambda i: (0, 0)),
            pl.BlockSpec((1, D), lambda i: (0, 0)),
            pl.BlockSpec((1, D), lambda i: (0, 0)),
            pl.BlockSpec((1, D), lambda i: (0, 0)),
        ],
        out_specs=pl.BlockSpec((blk, D), lambda i: (i, 0)),
        out_shape=jax.ShapeDtypeStruct((S, D), jnp.float32),
    )(flat, br, sr, W_g1, b_g1, w20, w21, b_g2, W_out, b_out, ln_g, ln_b)


def kernel(x, W_pred, buffer_mem, buffer_strength, store_keys, store_values,
           W_g1, b_g1, W_g2, b_g2, W_out, b_out, ln_g, ln_b,
           write_idx, store_idx):
    B, S, D = x.shape
    BUF = buffer_mem.shape[0]
    STORE = store_keys.shape[0]
    NCH = D // 16

    flat = x.reshape(S, D)
    write_idx = write_idx.astype(jnp.int32)
    store_idx = store_idx.astype(jnp.int32)

    uT, dst_idx, msk, bias = _prep(flat, W_pred, write_idx,
                                   buffer_strength, BUF, S)
    zeros = jnp.zeros((STORE, 16), jnp.float32)
    nb2 = _sc_scatter_nb(flat, dst_idx, BUF, D, S)
    delta = _sc_delta(uT, store_idx, zeros, STORE, D, S)
    br = _buffer_attn(flat, nb2, buffer_mem, msk, bias)
    sr = _store_attn(flat, store_keys, store_values, delta)
    out = _mlp_out(flat, br, sr, W_g1, b_g1.reshape(1, D),
                   W_g2[:, 0].reshape(1, D), W_g2[:, 1].reshape(1, D),
                   b_g2, W_out, b_out.reshape(1, D), ln_g.reshape(1, D),
                   ln_b.reshape(1, D))
    return out.reshape(B, S, D)
